# baseline (speedup 1.0000x reference)
import functools

import jax
import jax.numpy as jnp
from jax import lax
from jax.experimental import pallas as pl
from jax.experimental.pallas import tpu as pltpu
from jax.experimental.pallas import tpu_sc as plsc

D = 64
N_S = 883
N_T = 12
B = 1024
EPS = 1e-5

NW = 32
B_PER_W = B // NW

S_CHUNKS = ((0, 256), (256, 256), (512, 256), (768, 115))
S_BUF = 256 * D

T_ROW = N_T * D
T_BB = 8
T_BUF = T_BB * T_ROW


def _rsqrt_nr(a):
    i = lax.bitcast_convert_type(a, jnp.int32)
    i = jnp.int32(0x5F3759DF) - lax.shift_right_arithmetic(i, 1)
    y = lax.bitcast_convert_type(i, jnp.float32)
    for _ in range(3):
        y = y * (1.5 - 0.5 * a * y * y)
    return y


def _ln_rows(buf, bases, tab, tbases, g, bt):
    n = len(bases)
    ys = [None] * n
    rs = [None] * n
    for r in range(n):
        y = []
        for j in range(4):
            x = buf[pl.ds(bases[r] + j * 16, 16)]
            t = tab[pl.ds(tbases[r] + j * 16, 16)]
            y.append(x + t)
        ys[r] = y
    for r in range(n):
        y = ys[r]
        s = (y[0] + y[1]) + (y[2] + y[3])
        q = (y[0] * y[0] + y[1] * y[1]) + (y[2] * y[2] + y[3] * y[3])
        ssum = jnp.sum(s)
        qsum = jnp.sum(q)
        sv = jnp.broadcast_to(ssum, (16,))
        qv = jnp.broadcast_to(qsum, (16,))
        mean = sv * (1.0 / 64.0)
        var = qv * (1.0 / 64.0) - mean * mean
        rs[r] = (mean, _rsqrt_nr(var + EPS))
    for r in range(n):
        y = ys[r]
        mean, rinv = rs[r]
        for j in range(4):
            out = (y[j] - mean) * (rinv * g[j]) + bt[j]
            buf[pl.ds(bases[r] + j * 16, 16)] = out


def _sc_body(s_x, t_x, tab_s, tab_t, g_s, b_s, g_t, b_t,
             s_out, t_out,
             xbuf, tabbuf, tbuf, ttabbuf, gbbuf):
    wid = lax.axis_index("s") * 2 + lax.axis_index("c")
    b0 = wid * B_PER_W

    pltpu.sync_copy(g_s, gbbuf.at[pl.ds(0, D)])
    pltpu.sync_copy(b_s, gbbuf.at[pl.ds(D, D)])
    pltpu.sync_copy(g_t, gbbuf.at[pl.ds(2 * D, D)])
    pltpu.sync_copy(b_t, gbbuf.at[pl.ds(3 * D, D)])
    gs = [gbbuf[pl.ds(j * 16, 16)] for j in range(4)]
    bs = [gbbuf[pl.ds(D + j * 16, 16)] for j in range(4)]
    gt = [gbbuf[pl.ds(2 * D + j * 16, 16)] for j in range(4)]
    bt = [gbbuf[pl.ds(3 * D + j * 16, 16)] for j in range(4)]

    for (v0, c) in S_CHUNKS:
        pltpu.sync_copy(tab_s.at[pl.ds(v0 * D, c * D)], tabbuf.at[pl.ds(0, c * D)])

        def s_batch(bl, _, v0=v0, c=c):
            off = (b0 + bl) * (N_S * D) + v0 * D
            pltpu.sync_copy(s_x.at[pl.ds(off, c * D)], xbuf.at[pl.ds(0, c * D)])

            def rows4(i, _):
                base = i * (4 * D)
                _ln_rows(xbuf, [base + r * D for r in range(4)],
                         tabbuf, [base + r * D for r in range(4)], gs, bs)
                return 0

            lax.fori_loop(0, c // 4, rows4, 0)
            tail = c - (c // 4) * 4
            if tail:
                t0 = (c // 4) * 4 * D
                _ln_rows(xbuf, [t0 + r * D for r in range(tail)],
                         tabbuf, [t0 + r * D for r in range(tail)], gs, bs)
            pltpu.sync_copy(xbuf.at[pl.ds(0, c * D)], s_out.at[pl.ds(off, c * D)])
            return 0

        lax.fori_loop(0, B_PER_W, s_batch, 0)

    pltpu.sync_copy(tab_t, ttabbuf)

    def t_chunk(ci, _):
        off = (b0 + ci * T_BB) * T_ROW
        pltpu.sync_copy(t_x.at[pl.ds(off, T_BUF)], tbuf)

        def t_batch(q, _):
            qb = q * T_ROW
            _ln_rows(tbuf, [qb + v * D for v in range(N_T)],
                     ttabbuf, [v * D for v in range(N_T)], gt, bt)
            return 0

        lax.fori_loop(0, T_BB, t_batch, 0)
        pltpu.sync_copy(tbuf, t_out.at[pl.ds(off, T_BUF)])
        return 0

    lax.fori_loop(0, B_PER_W // T_BB, t_chunk, 0)


@jax.jit
def _run(s_x, t_x, tab_s, tab_t, g_s, b_s, g_t, b_t):
    mesh = plsc.VectorSubcoreMesh(core_axis_name="c", subcore_axis_name="s")
    kern = pl.kernel(
        _sc_body,
        out_type=[
            jax.ShapeDtypeStruct((B * N_S * D,), jnp.float32),
            jax.ShapeDtypeStruct((B * N_T * D,), jnp.float32),
        ],
        mesh=mesh,
        compiler_params=pltpu.CompilerParams(needs_layout_passes=False),
        scratch_types=[
            pltpu.VMEM((S_BUF,), jnp.float32),
            pltpu.VMEM((S_BUF,), jnp.float32),
            pltpu.VMEM((T_BUF,), jnp.float32),
            pltpu.VMEM((T_ROW,), jnp.float32),
            pltpu.VMEM((4 * D,), jnp.float32),
        ],
    )
    return kern(
        s_x.reshape(-1), t_x.reshape(-1), tab_s.reshape(-1), tab_t.reshape(-1),
        g_s, b_s, g_t, b_t,
    )


def kernel(s_x, t_x, pos_s_table, pos_t_table, gamma_s, beta_s, gamma_t, beta_t):
    s_flat, t_flat = _run(s_x, t_x, pos_s_table, pos_t_table,
                          gamma_s, beta_s, gamma_t, beta_t)
    return (s_flat.reshape(B, N_S, D), t_flat.reshape(B, N_T, D))


# speedup vs baseline: 1.1072x; 1.1072x over previous
import functools

import jax
import jax.numpy as jnp
from jax import lax
from jax.experimental import pallas as pl
from jax.experimental.pallas import tpu as pltpu
from jax.experimental.pallas import tpu_sc as plsc

D = 64
N_S = 883
N_T = 12
B = 1024
EPS = 1e-5

NW = 32
B_PER_W = B // NW

S_CHUNKS = ((0, 256), (256, 256), (512, 256), (768, 115))
S_BUF = 256 * D

T_ROW = N_T * D
T_BB = 8
T_BUF = T_BB * T_ROW


def _rsqrt_nr(a):
    i = lax.bitcast_convert_type(a, jnp.int32)
    i = jnp.int32(0x5F3759DF) - lax.shift_right_arithmetic(i, 1)
    y = lax.bitcast_convert_type(i, jnp.float32)
    for _ in range(3):
        y = y * (1.5 - 0.5 * a * y * y)
    return y


def _ln_rows(buf, bases, tab, tbases, g, bt):
    n = len(bases)
    ys = [None] * n
    rs = [None] * n
    for r in range(n):
        y = []
        for j in range(4):
            x = buf[pl.ds(bases[r] + j * 16, 16)]
            t = tab[pl.ds(tbases[r] + j * 16, 16)]
            y.append(x + t)
        ys[r] = y
    for r in range(n):
        y = ys[r]
        s = (y[0] + y[1]) + (y[2] + y[3])
        q = (y[0] * y[0] + y[1] * y[1]) + (y[2] * y[2] + y[3] * y[3])
        ssum = jnp.sum(s)
        qsum = jnp.sum(q)
        sv = jnp.broadcast_to(ssum, (16,))
        qv = jnp.broadcast_to(qsum, (16,))
        mean = sv * (1.0 / 64.0)
        var = qv * (1.0 / 64.0) - mean * mean
        rs[r] = (mean, _rsqrt_nr(var + EPS))
    for r in range(n):
        y = ys[r]
        mean, rinv = rs[r]
        for j in range(4):
            out = (y[j] - mean) * (rinv * g[j]) + bt[j]
            buf[pl.ds(bases[r] + j * 16, 16)] = out


def _compute_s_chunk(buf, tabbuf, c, gs, bs):
    @plsc.parallel_loop(0, c // 4, unroll=2)
    def rows4(i):
        base = i * (4 * D)
        _ln_rows(buf, [base + r * D for r in range(4)],
                 tabbuf, [base + r * D for r in range(4)], gs, bs)

    tail = c - (c // 4) * 4
    if tail:
        t0 = (c // 4) * 4 * D
        _ln_rows(buf, [t0 + r * D for r in range(tail)],
                 tabbuf, [t0 + r * D for r in range(tail)], gs, bs)


def _sc_body(s_x, t_x, tab_s, tab_t, g_s, b_s, g_t, b_t,
             s_out, t_out,
             xbufa, xbufb, tabbuf, tbuf, ttabbuf, gbbuf,
             sem_ain, sem_aout, sem_bin, sem_bout):
    wid = lax.axis_index("s") * 2 + lax.axis_index("c")
    b0 = wid * B_PER_W

    pltpu.sync_copy(g_s, gbbuf.at[pl.ds(0, D)])
    pltpu.sync_copy(b_s, gbbuf.at[pl.ds(D, D)])
    pltpu.sync_copy(g_t, gbbuf.at[pl.ds(2 * D, D)])
    pltpu.sync_copy(b_t, gbbuf.at[pl.ds(3 * D, D)])
    gs = [gbbuf[pl.ds(j * 16, 16)] for j in range(4)]
    bs = [gbbuf[pl.ds(D + j * 16, 16)] for j in range(4)]
    gt = [gbbuf[pl.ds(2 * D + j * 16, 16)] for j in range(4)]
    bt = [gbbuf[pl.ds(3 * D + j * 16, 16)] for j in range(4)]

    for (v0, c) in S_CHUNKS:
        n = c * D
        pltpu.sync_copy(tab_s.at[pl.ds(v0 * D, n)], tabbuf.at[pl.ds(0, n)])

        def off(bl, v0=v0):
            return (b0 + bl) * (N_S * D) + v0 * D

        def in_cp(bl, buf, sem, v0=v0, n=n):
            return pltpu.make_async_copy(
                s_x.at[pl.ds(off(bl, v0), n)], buf.at[pl.ds(0, n)], sem)

        def out_cp(bl, buf, sem, v0=v0, n=n):
            return pltpu.make_async_copy(
                buf.at[pl.ds(0, n)], s_out.at[pl.ds(off(bl, v0), n)], sem)

        in_cp(0, xbufa, sem_ain).start()

        def pair(i, _, v0=v0, c=c, n=n):
            ba, bb = 2 * i, 2 * i + 1

            @pl.when(i > 0)
            def _():
                out_cp(bb, xbufb, sem_bout).wait()

            in_cp(bb, xbufb, sem_bin).start()
            in_cp(ba, xbufa, sem_ain).wait()
            _compute_s_chunk(xbufa, tabbuf, c, gs, bs)
            out_cp(ba, xbufa, sem_aout).start()
            in_cp(bb, xbufb, sem_bin).wait()
            _compute_s_chunk(xbufb, tabbuf, c, gs, bs)
            out_cp(bb, xbufb, sem_bout).start()

            @pl.when(i < B_PER_W // 2 - 1)
            def _():
                out_cp(ba, xbufa, sem_aout).wait()
                in_cp(ba + 2, xbufa, sem_ain).start()

            return 0

        lax.fori_loop(0, B_PER_W // 2, pair, 0)
        out_cp(B_PER_W - 2, xbufa, sem_aout).wait()
        out_cp(B_PER_W - 1, xbufb, sem_bout).wait()

    pltpu.sync_copy(tab_t, ttabbuf)

    def t_chunk(ci, _):
        toff = (b0 + ci * T_BB) * T_ROW
        pltpu.sync_copy(t_x.at[pl.ds(toff, T_BUF)], tbuf)

        @plsc.parallel_loop(0, T_BB)
        def t_batch(q):
            qb = q * T_ROW
            for half in range(2):
                _ln_rows(tbuf, [qb + (half * 6 + v) * D for v in range(6)],
                         ttabbuf, [(half * 6 + v) * D for v in range(6)],
                         gt, bt)

        pltpu.sync_copy(tbuf, t_out.at[pl.ds(toff, T_BUF)])
        return 0

    lax.fori_loop(0, B_PER_W // T_BB, t_chunk, 0)


@jax.jit
def _run(s_x, t_x, tab_s, tab_t, g_s, b_s, g_t, b_t):
    mesh = plsc.VectorSubcoreMesh(core_axis_name="c", subcore_axis_name="s")
    kern = pl.kernel(
        _sc_body,
        out_type=[
            jax.ShapeDtypeStruct((B * N_S * D,), jnp.float32),
            jax.ShapeDtypeStruct((B * N_T * D,), jnp.float32),
        ],
        mesh=mesh,
        compiler_params=pltpu.CompilerParams(needs_layout_passes=False),
        scratch_types=[
            pltpu.VMEM((S_BUF,), jnp.float32),
            pltpu.VMEM((S_BUF,), jnp.float32),
            pltpu.VMEM((S_BUF,), jnp.float32),
            pltpu.VMEM((T_BUF,), jnp.float32),
            pltpu.VMEM((T_ROW,), jnp.float32),
            pltpu.VMEM((4 * D,), jnp.float32),
            pltpu.SemaphoreType.DMA,
            pltpu.SemaphoreType.DMA,
            pltpu.SemaphoreType.DMA,
            pltpu.SemaphoreType.DMA,
        ],
    )
    return kern(
        s_x.reshape(-1), t_x.reshape(-1), tab_s.reshape(-1), tab_t.reshape(-1),
        g_s, b_s, g_t, b_t,
    )


def kernel(s_x, t_x, pos_s_table, pos_t_table, gamma_s, beta_s, gamma_t, beta_t):
    s_flat, t_flat = _run(s_x, t_x, pos_s_table, pos_t_table,
                          gamma_s, beta_s, gamma_t, beta_t)
    return (s_flat.reshape(B, N_S, D), t_flat.reshape(B, N_T, D))
